# baseline (speedup 1.0000x reference)
import jax
import jax.numpy as jnp
from jax import lax
from jax.experimental import pallas as pl
from jax.experimental.pallas import tpu as pltpu

_MAX_TILE = 2048
_N_CORES = 2


def _bilinear_kernel(q_ref, wq_ref, bq_ref, wque_ref, bque_ref,
                     s_hbm, ws_hbm, bs_ref, wsup_hbm, bsup_ref,
                     out_ref,
                     wq16_s, wque16_s, bque_s, fs16_s, fq16_s,
                     s32_s, ws32_s, wsup32_s, sems):
    j = pl.program_id(1)
    num_pairs, _, hid = wque_ref.shape

    cp_s = pltpu.make_async_copy(s_hbm, s32_s, sems.at[0])
    cp_ws = pltpu.make_async_copy(ws_hbm, ws32_s, sems.at[1])
    cp_wsup = pltpu.make_async_copy(wsup_hbm, wsup32_s, sems.at[2])

    @pl.when(j == 0)
    def _():
        cp_s.start()
        cp_ws.start()
        cp_wsup.start()
        for pp in range(num_pairs):
            sl = slice(pp * hid, (pp + 1) * hid)
            wque16_s[:, sl] = wque_ref[pp].astype(jnp.bfloat16)
            bque_s[:, sl] = bque_ref[pp:pp + 1, :]
        wq16_s[...] = wq_ref[...].astype(jnp.bfloat16)

    tq = q_ref.shape[1]
    n_sub = max(1, tq // 512)
    w = tq // n_sub
    for h in range(n_sub):
        rows = slice(h * w, (h + 1) * w)
        q16 = q_ref[:, rows].astype(jnp.bfloat16)
        que = lax.dot_general(q16, wq16_s[...], (((0,), (0,)), ((), ())),
                              preferred_element_type=jnp.float32)
        que = que + bq_ref[...]
        f_q = jnp.dot(que.astype(jnp.bfloat16), wque16_s[...],
                      preferred_element_type=jnp.float32) + bque_s[...]
        fq16_s[rows, :] = f_q.astype(jnp.bfloat16)

    @pl.when(j == 0)
    def _():
        cp_s.wait()
        cp_ws.wait()
        cp_wsup.wait()
        ns = float(s32_s.shape[1])
        s_sum = jnp.sum(s32_s[...], axis=1, keepdims=True)
        sup = lax.dot_general(s_sum, ws32_s[...], (((0,), (0,)), ((), ())),
                              preferred_element_type=jnp.float32)
        sup = sup + ns * bs_ref[...]
        for pp in range(num_pairs):
            f_sp = jnp.dot(sup, wsup32_s[pp],
                           preferred_element_type=jnp.float32) \
                + ns * bsup_ref[pp:pp + 1, :]
            fs16_s[:, pp * hid:(pp + 1) * hid] = f_sp.astype(jnp.bfloat16)

    for h in range(n_sub):
        rows = slice(h * w, (h + 1) * w)
        logits = lax.dot_general(fs16_s[...], fq16_s[rows, :],
                                 (((1,), (1,)), ((), ())),
                                 preferred_element_type=jnp.float32)
        out_ref[:, rows] = 1.0 / (1.0 + jnp.exp(-logits))


def _pick_tile(nq, max_tile=_MAX_TILE):
    if nq <= max_tile or nq % 128 != 0:
        return nq
    t = max_tile - (max_tile % 128)
    while t >= 128:
        if nq % t == 0:
            return t
        t -= 128
    return nq


def kernel(query_emb, support_emb, wq, bq, ws, bs, wque, bque, wsup, bsup):
    din, nq = query_emb.shape
    _, ns = support_emb.shape
    p, dout, hid = wque.shape
    ph = p * hid

    bq2 = bq.reshape(1, dout)
    bs2 = bs.reshape(1, dout)

    tq = _pick_tile(nq)
    n_tiles = nq // tq
    n_cores = _N_CORES if n_tiles % _N_CORES == 0 else 1
    spc = n_tiles // n_cores

    out = pl.pallas_call(
        _bilinear_kernel,
        out_shape=jax.ShapeDtypeStruct((1, nq), jnp.float32),
        grid=(n_cores, spc),
        in_specs=[
            pl.BlockSpec((din, tq), lambda i, j: (0, i * spc + j)),
            pl.BlockSpec((din, dout), lambda i, j: (0, 0)),
            pl.BlockSpec((1, dout), lambda i, j: (0, 0)),
            pl.BlockSpec((p, dout, hid), lambda i, j: (0, 0, 0)),
            pl.BlockSpec((p, hid), lambda i, j: (0, 0)),
            pl.BlockSpec(memory_space=pl.ANY),
            pl.BlockSpec(memory_space=pl.ANY),
            pl.BlockSpec((1, dout), lambda i, j: (0, 0)),
            pl.BlockSpec(memory_space=pl.ANY),
            pl.BlockSpec((p, hid), lambda i, j: (0, 0)),
        ],
        out_specs=pl.BlockSpec((1, tq), lambda i, j: (0, i * spc + j)),
        scratch_shapes=[
            pltpu.VMEM((din, dout), jnp.bfloat16),
            pltpu.VMEM((dout, ph), jnp.bfloat16),
            pltpu.VMEM((1, ph), jnp.float32),
            pltpu.VMEM((1, ph), jnp.bfloat16),
            pltpu.VMEM((tq, ph), jnp.bfloat16),
            pltpu.VMEM((din, ns), jnp.float32),
            pltpu.VMEM((din, dout), jnp.float32),
            pltpu.VMEM((p, dout, hid), jnp.float32),
            pltpu.SemaphoreType.DMA((3,)),
        ],
        compiler_params=pltpu.CompilerParams(
            dimension_semantics=("parallel", "arbitrary")),
    )(query_emb, wq, bq2, wque, bque,
      support_emb, ws, bs2, wsup, bsup)

    return out.reshape(nq)
